# baseline (speedup 1.0000x reference)
import functools

import numpy as np
import jax
import jax.numpy as jnp
from jax import lax
from jax.experimental import pallas as pl
from jax.experimental.pallas import tpu as pltpu
from jax.experimental.pallas import tpu_sc as plsc

_OUT_DIM = 5
_NBASIS = _OUT_DIM * 16

_rng = np.random.RandomState(0)
_HASH_A = [int(v) for v in _rng.randint(1, 2**31 - 1, size=(_OUT_DIM,)).astype(np.int32)]
_HASH_B = [int(v) for v in _rng.randint(0, 2**31 - 1, size=(_OUT_DIM,)).astype(np.int32)]

_BATCH = 16384
_NCORES = 1
_NWORKERS = 16 * _NCORES
_BPW = _BATCH // _NWORKERS
_LANES = 16
_CHUNKS = _BPW // _LANES


def _embed_body(x_hbm, v_hbm, out_hbm, x_v, tab_v, out_v):
    wid = lax.axis_index("s") * _NCORES + lax.axis_index("c")
    base = wid * _BPW
    pltpu.sync_copy(x_hbm.at[pl.ds(base, _BPW)], x_v)
    pltpu.sync_copy(v_hbm, tab_v)
    lane = lax.iota(jnp.int32, 16)

    def chunk(i, carry):
        xi = x_v[pl.ds(i * _LANES, _LANES)]
        embeds = []
        for d in range(_OUT_DIM):
            h = xi * _HASH_A[d] + _HASH_B[d]
            r = lax.rem(h, _NBASIS)
            r = jnp.where(r < 0, r + _NBASIS, r)
            embeds.append(plsc.load_gather(tab_v, [r]))
        total = embeds[0]
        for d in range(1, _OUT_DIM):
            total = total + embeds[d]
        scale = jnp.float32(_OUT_DIM) / total
        row = i * _LANES + lane
        for d in range(_OUT_DIM):
            col = jnp.full((16,), d, jnp.int32)
            plsc.store_scatter(out_v, [row, col], embeds[d] * scale)
        return carry

    lax.fori_loop(0, _CHUNKS, chunk, 0)
    pltpu.sync_copy(out_v, out_hbm.at[pl.ds(base, _BPW)])


@jax.jit
def kernel(x, V):
    mesh = plsc.VectorSubcoreMesh(
        core_axis_name="c", subcore_axis_name="s", num_cores=_NCORES
    )
    run = functools.partial(
        pl.kernel,
        mesh=mesh,
        out_type=jax.ShapeDtypeStruct((_BATCH, _OUT_DIM), jnp.float32),
        compiler_params=pltpu.CompilerParams(
            needs_layout_passes=False, use_tc_tiling_on_sc=False
        ),
        scratch_types=[
            pltpu.VMEM((_BPW,), jnp.int32),
            pltpu.VMEM((_NBASIS,), jnp.float32),
            pltpu.VMEM((_BPW, _OUT_DIM), jnp.float32),
        ],
    )(_embed_body)
    return run(x, V)


# speedup vs baseline: 1.6614x; 1.6614x over previous
import functools

import numpy as np
import jax
import jax.numpy as jnp
from jax import lax
from jax.experimental import pallas as pl
from jax.experimental.pallas import tpu as pltpu
from jax.experimental.pallas import tpu_sc as plsc

_OUT_DIM = 5
_NBASIS = _OUT_DIM * 16

_rng = np.random.RandomState(0)
_HASH_A = [int(v) for v in _rng.randint(1, 2**31 - 1, size=(_OUT_DIM,)).astype(np.int32)]
_HASH_B = [int(v) for v in _rng.randint(0, 2**31 - 1, size=(_OUT_DIM,)).astype(np.int32)]

_BATCH = 16384
_NCORES = 2
_NWORKERS = 16 * _NCORES
_BPW = _BATCH // _NWORKERS
_LANES = 16
_CHUNKS = _BPW // _LANES


def _embed_body(x_hbm, v_hbm, out_hbm, x_v, tab_v, out_v):
    wid = lax.axis_index("s") * _NCORES + lax.axis_index("c")
    base = wid * _BPW
    pltpu.sync_copy(x_hbm.at[pl.ds(base, _BPW)], x_v)
    pltpu.sync_copy(v_hbm, tab_v)

    def chunk(i, carry):
        xi = x_v[pl.ds(i * _LANES, _LANES)]
        embeds = []
        for d in range(_OUT_DIM):
            h = xi * _HASH_A[d] + _HASH_B[d]
            r = lax.rem(h, _NBASIS)
            r = jnp.where(r < 0, r + _NBASIS, r)
            embeds.append(plsc.load_gather(tab_v, [r]))
        total = embeds[0]
        for d in range(1, _OUT_DIM):
            total = total + embeds[d]
        scale = jnp.float32(_OUT_DIM) / total
        for d in range(_OUT_DIM):
            out_v[d, pl.ds(i * _LANES, _LANES)] = embeds[d] * scale
        return carry

    lax.fori_loop(0, _CHUNKS, chunk, 0)
    pltpu.sync_copy(out_v, out_hbm.at[:, pl.ds(base, _BPW)])


@jax.jit
def kernel(x, V):
    mesh = plsc.VectorSubcoreMesh(
        core_axis_name="c", subcore_axis_name="s", num_cores=_NCORES
    )
    run = functools.partial(
        pl.kernel,
        mesh=mesh,
        out_type=jax.ShapeDtypeStruct((_OUT_DIM, _BATCH), jnp.float32),
        compiler_params=pltpu.CompilerParams(needs_layout_passes=False),
        scratch_types=[
            pltpu.VMEM((_BPW,), jnp.int32),
            pltpu.VMEM((_NBASIS,), jnp.float32),
            pltpu.VMEM((_OUT_DIM, _BPW), jnp.float32),
        ],
    )(_embed_body)
    return run(x, V).T
